# baseline (device time: 62773 ns/iter reference)
import jax
import jax.numpy as jnp
from jax import lax
from jax.experimental import pallas as pl
from jax.experimental.pallas import tpu as pltpu

T = 1024
D = 1024
F = 2048
E = 8
TSH = 512
ESH = 4
NC = 4
CH = TSH // NC
NEG = -1e30


def _mm(a, b, precision=None):
    return lax.dot_general(
        a, b, (((1,), (0,)), ((), ())),
        preferred_element_type=jnp.float32, precision=precision,
    )


def kernel(x, router, W1, W2):
    def body(x_ref, r_ref, w1_hbm, w2_hbm, out_ref,
             xsend, xrecv, wsend, wrecv, rfull, stg1, stg2, w1buf, w2buf,
             part_other, part_mine, precv, bsend, brecv,
             send_sems, recv_sems, load_sems):
        p = lax.axis_index("x")
        q = lax.axis_index("y")
        xn = (1 - p, q)
        yn = (p, 1 - q)

        le0 = 2 * q
        le1 = 2 * q + 1
        ld10 = pltpu.make_async_copy(w1_hbm.at[le0], stg1, load_sems.at[0])
        ld20 = pltpu.make_async_copy(w2_hbm.at[le0], stg2, load_sems.at[1])
        ld10.start()
        ld20.start()
        cp_rl = pltpu.make_async_copy(r_ref, rfull.at[p], load_sems.at[4])
        cp_rl.start()

        bar = pltpu.get_barrier_semaphore()
        for nbr in (xn, yn):
            pl.semaphore_signal(bar, inc=1, device_id=nbr,
                                device_id_type=pl.DeviceIdType.MESH)
        pl.semaphore_wait(bar, 2)

        cp_r = pltpu.make_async_remote_copy(
            src_ref=r_ref, dst_ref=rfull.at[p],
            send_sem=send_sems.at[0], recv_sem=recv_sems.at[0],
            device_id=xn, device_id_type=pl.DeviceIdType.MESH)
        cp_r.start()
        xsend[...] = x_ref[...].astype(jnp.bfloat16)
        cp_x = pltpu.make_async_remote_copy(
            src_ref=xsend, dst_ref=xrecv,
            send_sem=send_sems.at[1], recv_sem=recv_sems.at[1],
            device_id=xn, device_id_type=pl.DeviceIdType.MESH)
        cp_x.start()

        cp_r.wait()
        cp_rl.wait()
        idx4 = lax.broadcasted_iota(jnp.int32, (TSH, ESH), 1)
        g0 = _mm(x_ref[...], rfull[0], lax.Precision.HIGHEST)
        g1 = _mm(x_ref[...], rfull[1], lax.Precision.HIGHEST)
        m0 = jnp.max(g0, axis=1, keepdims=True)
        i0 = jnp.min(jnp.where(g0 >= m0, idx4, ESH), axis=1, keepdims=True)
        m1 = jnp.max(g1, axis=1, keepdims=True)
        i1 = jnp.min(jnp.where(g1 >= m1, idx4, ESH), axis=1, keepdims=True)
        t1 = jnp.maximum(m0, m1)
        it1 = jnp.where(m0 >= m1, i0, i1 + 4)
        g0m = jnp.where((it1 < 4) & (idx4 == it1), NEG, g0)
        g1m = jnp.where((it1 >= 4) & (idx4 == it1 - 4), NEG, g1)
        m0b = jnp.max(g0m, axis=1, keepdims=True)
        i0b = jnp.min(jnp.where(g0m >= m0b, idx4, ESH), axis=1, keepdims=True)
        m1b = jnp.max(g1m, axis=1, keepdims=True)
        i1b = jnp.min(jnp.where(g1m >= m1b, idx4, ESH), axis=1, keepdims=True)
        t2 = jnp.maximum(m0b, m1b)
        it2 = jnp.where(m0b >= m1b, i0b, i1b + 4)
        e2 = jnp.exp(t2 - t1)
        wa = 1.0 / (1.0 + e2)
        wb = e2 / (1.0 + e2)

        def ew(eg):
            return (jnp.where(it1 == eg, wa, 0.0)
                    + jnp.where(it2 == eg, wb, 0.0))

        wsend[:, 0:1] = ew(4 * (1 - p) + le0).astype(jnp.bfloat16)
        wsend[:, 1:2] = ew(4 * (1 - p) + le1).astype(jnp.bfloat16)
        cp_w = pltpu.make_async_remote_copy(
            src_ref=wsend, dst_ref=wrecv,
            send_sem=send_sems.at[2], recv_sem=recv_sems.at[2],
            device_id=xn, device_id_type=pl.DeviceIdType.MESH)
        cp_w.start()

        def ffn(xb, slot):
            h = jnp.maximum(_mm(xb, w1buf[slot]), 0.0).astype(jnp.bfloat16)
            return _mm(h, w2buf[slot])

        xmb = x_ref[...].astype(jnp.bfloat16)
        ld10.wait()
        w1buf[0, :, :] = stg1[...].astype(jnp.bfloat16)
        ld20.wait()
        w2buf[0, :, :] = stg2[...].astype(jnp.bfloat16)
        ld11 = pltpu.make_async_copy(w1_hbm.at[le1], stg1, load_sems.at[0])
        ld21 = pltpu.make_async_copy(w2_hbm.at[le1], stg2, load_sems.at[1])
        ld11.start()
        ld21.start()

        part_mine[...] = ffn(xmb, 0) * ew(4 * p + le0)

        ld11.wait()
        w1buf[1, :, :] = stg1[...].astype(jnp.bfloat16)
        ld21.wait()
        w2buf[1, :, :] = stg2[...].astype(jnp.bfloat16)

        cp_x.wait()
        cp_w.wait()
        cps_a = []
        for k in range(NC):
            ck = pl.ds(k * CH, CH)
            xok = xrecv[ck, :]
            ck_part = (ffn(xok, 0) * wrecv[ck, 0:1].astype(jnp.float32)
                       + ffn(xok, 1) * wrecv[ck, 1:2].astype(jnp.float32))
            part_other[ck, :] = ck_part.astype(jnp.bfloat16)
            cp_a = pltpu.make_async_remote_copy(
                src_ref=part_other.at[ck], dst_ref=precv.at[ck],
                send_sem=send_sems.at[3 + k], recv_sem=recv_sems.at[3 + k],
                device_id=xn, device_id_type=pl.DeviceIdType.MESH)
            cp_a.start()
            cps_a.append(cp_a)

        part_mine[...] = part_mine[...] + ffn(xmb, 1) * ew(4 * p + le1)

        cps_b = []
        for k in range(NC):
            ck = pl.ds(k * CH, CH)
            cps_a[k].wait()
            s = part_mine[ck, :] + precv[ck, :].astype(jnp.float32)
            out_ref[ck, :] = s
            bsend[ck, :] = s.astype(jnp.bfloat16)
            cp_b = pltpu.make_async_remote_copy(
                src_ref=bsend.at[ck], dst_ref=brecv.at[ck],
                send_sem=send_sems.at[3 + NC + k],
                recv_sem=recv_sems.at[3 + NC + k],
                device_id=yn, device_id_type=pl.DeviceIdType.MESH)
            cp_b.start()
            cps_b.append(cp_b)
        for k in range(NC):
            ck = pl.ds(k * CH, CH)
            cps_b[k].wait()
            out_ref[ck, :] = out_ref[ck, :] + brecv[ck, :].astype(jnp.float32)

    return pl.pallas_call(
        body,
        out_shape=jax.ShapeDtypeStruct((TSH, D), jnp.float32),
        in_specs=[
            pl.BlockSpec(memory_space=pltpu.VMEM),
            pl.BlockSpec(memory_space=pltpu.VMEM),
            pl.BlockSpec(memory_space=pl.ANY),
            pl.BlockSpec(memory_space=pl.ANY),
        ],
        out_specs=pl.BlockSpec(memory_space=pltpu.VMEM),
        scratch_shapes=[
            pltpu.VMEM((TSH, D), jnp.bfloat16),
            pltpu.VMEM((TSH, D), jnp.bfloat16),
            pltpu.VMEM((TSH, 128), jnp.bfloat16),
            pltpu.VMEM((TSH, 128), jnp.bfloat16),
            pltpu.VMEM((2, D, ESH), jnp.float32),
            pltpu.VMEM((D, F), jnp.float32),
            pltpu.VMEM((F, D), jnp.float32),
            pltpu.VMEM((2, D, F), jnp.bfloat16),
            pltpu.VMEM((2, F, D), jnp.bfloat16),
            pltpu.VMEM((TSH, D), jnp.bfloat16),
            pltpu.VMEM((TSH, D), jnp.float32),
            pltpu.VMEM((TSH, D), jnp.bfloat16),
            pltpu.VMEM((TSH, D), jnp.bfloat16),
            pltpu.VMEM((TSH, D), jnp.bfloat16),
            pltpu.SemaphoreType.DMA((3 + 2 * NC,)),
            pltpu.SemaphoreType.DMA((3 + 2 * NC,)),
            pltpu.SemaphoreType.DMA((5,)),
        ],
        compiler_params=pltpu.CompilerParams(
            collective_id=0, vmem_limit_bytes=60 * 1024 * 1024
        ),
    )(x, router, W1, W2)
